# baseline (speedup 1.0000x reference)
import jax
import jax.numpy as jnp
from jax.experimental import pallas as pl
from jax.experimental.pallas import tpu as pltpu

B, C, H, W = 64, 768, 24, 24
HW = H * W
E = 8
BB = 8
NSTEPS = B // BB
KS = 4
HK = HW // KS
NEG_INF = float("-inf")


def _gate_kernel(x_ref, w0_ref, b0_ref, w1_ref, b1_ref, out_ref,
                 accm, accs):
    j = pl.program_id(0)
    k = pl.program_id(1)
    blk = x_ref[...]
    pmax = jnp.max(blk, axis=1)
    psum = jnp.sum(blk, axis=1)
    rows = pl.ds(j * BB, BB)

    @pl.when(k == 0)
    def _first():
        accm[rows, :] = pmax
        accs[rows, :] = psum

    @pl.when(k > 0)
    def _rest():
        accm[rows, :] = jnp.maximum(accm[rows, :], pmax)
        accs[rows, :] = accs[rows, :] + psum

    @pl.when((j == NSTEPS - 1) & (k == KS - 1))
    def _epilogue():
        pooled = accm[...] + accs[...] * (1.0 / HW)
        dn = (((1,), (1,)), ((), ()))
        h = jax.lax.dot_general(
            pooled, w0_ref[...], dn,
            preferred_element_type=jnp.float32) + b0_ref[...]
        h = jnp.where(h >= 0.0, h, 0.2 * h)
        z = jax.lax.dot_general(
            pooled, w1_ref[...], dn,
            preferred_element_type=jnp.float32) + b1_ref[...]
        noise = jnp.maximum(z, 0.0) + jnp.log1p(jnp.exp(-jnp.abs(z)))
        nmean = jnp.mean(noise, axis=1, keepdims=True)
        var = jnp.sum((noise - nmean) ** 2, axis=1, keepdims=True) / (E - 1)
        norm_noise = (noise - nmean) * jax.lax.rsqrt(var)
        scores = h + norm_noise
        ii = jax.lax.broadcasted_iota(jnp.int32, (B, E), 1)
        m1 = jnp.max(scores, axis=1, keepdims=True)
        i1 = jnp.min(jnp.where(scores == m1, ii, E), axis=1, keepdims=True)
        oh1 = ii == i1
        s2 = jnp.where(oh1, NEG_INF, scores)
        m2 = jnp.max(s2, axis=1, keepdims=True)
        i2 = jnp.min(jnp.where(s2 == m2, ii, E), axis=1, keepdims=True)
        mask = oh1 | (ii == i2)
        hm = jnp.where(mask, h, NEG_INF)
        mx = jnp.max(hm, axis=1, keepdims=True)
        e = jnp.where(mask, jnp.exp(h - mx), 0.0)
        gate = e / jnp.sum(e, axis=1, keepdims=True)
        out_ref[...] = gate.T


@jax.jit
def kernel(x, W0, b0, W1, b1):
    xt = jnp.transpose(x, (0, 2, 3, 1)).reshape(B, HW, C)
    gate_t = pl.pallas_call(
        _gate_kernel,
        grid=(NSTEPS, KS),
        in_specs=[
            pl.BlockSpec((BB, HK, C), lambda j, k: (j, k, 0)),
            pl.BlockSpec((E, C), lambda j, k: (0, 0)),
            pl.BlockSpec((1, E), lambda j, k: (0, 0)),
            pl.BlockSpec((E, C), lambda j, k: (0, 0)),
            pl.BlockSpec((1, E), lambda j, k: (0, 0)),
        ],
        out_specs=pl.BlockSpec((E, B), lambda j, k: (0, 0)),
        out_shape=jax.ShapeDtypeStruct((E, B), jnp.float32),
        scratch_shapes=[
            pltpu.VMEM((B, C), jnp.float32),
            pltpu.VMEM((B, C), jnp.float32),
        ],
    )(xt, W0, b0.reshape(1, E), W1, b1.reshape(1, E))
    return gate_t.T


# speedup vs baseline: 1.2183x; 1.2183x over previous
import jax
import jax.numpy as jnp
from jax.experimental import pallas as pl
from jax.experimental.pallas import tpu as pltpu

B, C, H, W = 64, 768, 24, 24
HW = H * W
E = 8
BB = 8
NSTEPS = B // BB
KS = 2
HK = HW // KS
NEG_INF = float("-inf")


def _gate_kernel(x_ref, w0_ref, b0_ref, w1_ref, b1_ref, out_ref,
                 accm, accs):
    j = pl.program_id(0)
    k = pl.program_id(1)
    blk = x_ref[...]
    pmax = jnp.max(blk, axis=1)
    psum = jnp.sum(blk, axis=1)
    rows = pl.ds(j * BB, BB)

    @pl.when(k == 0)
    def _first():
        accm[rows, :] = pmax
        accs[rows, :] = psum

    @pl.when(k > 0)
    def _rest():
        accm[rows, :] = jnp.maximum(accm[rows, :], pmax)
        accs[rows, :] = accs[rows, :] + psum

    @pl.when((j == NSTEPS - 1) & (k == KS - 1))
    def _epilogue():
        pooled = accm[...] + accs[...] * (1.0 / HW)
        dn = (((1,), (1,)), ((), ()))
        h = jax.lax.dot_general(
            pooled, w0_ref[...], dn,
            preferred_element_type=jnp.float32) + b0_ref[...]
        h = jnp.where(h >= 0.0, h, 0.2 * h)
        z = jax.lax.dot_general(
            pooled, w1_ref[...], dn,
            preferred_element_type=jnp.float32) + b1_ref[...]
        noise = jnp.maximum(z, 0.0) + jnp.log1p(jnp.exp(-jnp.abs(z)))
        nmean = jnp.mean(noise, axis=1, keepdims=True)
        var = jnp.sum((noise - nmean) ** 2, axis=1, keepdims=True) / (E - 1)
        norm_noise = (noise - nmean) * jax.lax.rsqrt(var)
        scores = h + norm_noise
        ii = jax.lax.broadcasted_iota(jnp.int32, (B, E), 1)
        m1 = jnp.max(scores, axis=1, keepdims=True)
        i1 = jnp.min(jnp.where(scores == m1, ii, E), axis=1, keepdims=True)
        oh1 = ii == i1
        s2 = jnp.where(oh1, NEG_INF, scores)
        m2 = jnp.max(s2, axis=1, keepdims=True)
        i2 = jnp.min(jnp.where(s2 == m2, ii, E), axis=1, keepdims=True)
        mask = oh1 | (ii == i2)
        hm = jnp.where(mask, h, NEG_INF)
        mx = jnp.max(hm, axis=1, keepdims=True)
        e = jnp.where(mask, jnp.exp(h - mx), 0.0)
        gate = e / jnp.sum(e, axis=1, keepdims=True)
        out_ref[...] = gate.T


@jax.jit
def kernel(x, W0, b0, W1, b1):
    xt = jnp.transpose(x, (0, 2, 3, 1)).reshape(B, HW, C)
    gate_t = pl.pallas_call(
        _gate_kernel,
        grid=(NSTEPS, KS),
        in_specs=[
            pl.BlockSpec((BB, HK, C), lambda j, k: (j, k, 0)),
            pl.BlockSpec((E, C), lambda j, k: (0, 0)),
            pl.BlockSpec((1, E), lambda j, k: (0, 0)),
            pl.BlockSpec((E, C), lambda j, k: (0, 0)),
            pl.BlockSpec((1, E), lambda j, k: (0, 0)),
        ],
        out_specs=pl.BlockSpec((E, B), lambda j, k: (0, 0)),
        out_shape=jax.ShapeDtypeStruct((E, B), jnp.float32),
        scratch_shapes=[
            pltpu.VMEM((B, C), jnp.float32),
            pltpu.VMEM((B, C), jnp.float32),
        ],
    )(xt, W0, b0.reshape(1, E), W1, b1.reshape(1, E))
    return gate_t.T
